# baseline (speedup 1.0000x reference)
import functools

import jax
import jax.numpy as jnp
from jax import lax
from jax.experimental import pallas as pl
from jax.experimental.pallas import tpu as pltpu
from jax.experimental.pallas import tpu_sc as plsc

B = 16384
D = 128
HALF = D // 2
L = 16
NC, NS = 2, 16
NW = NC * NS
BPW = B // NW
C = 128
CHUNKS = ((0, 32), (32, 96), (128, 128), (256, 128), (384, 128))

_mesh = plsc.VectorSubcoreMesh(core_axis_name="c", subcore_axis_name="s")


_PERM_DNUMS = lax.GatherDimensionNumbers(
    offset_dims=(), collapsed_slice_dims=(0,), start_index_map=(0,))


def _perm(v, idx):
    return lax.gather(v, idx[:, None], _PERM_DNUMS, (1,),
                      mode=lax.GatherScatterMode.PROMISE_IN_BOUNDS)


@functools.partial(
    pl.kernel,
    mesh=_mesh,
    out_type=jax.ShapeDtypeStruct((B,), jnp.float32),
    scratch_types=[
        pltpu.VMEM((BPW,), jnp.int32),
        pltpu.VMEM((BPW,), jnp.int32),
        pltpu.VMEM((BPW,), jnp.int32),
        pltpu.VMEM((2, C, D), jnp.float32),
        pltpu.VMEM((2, C, D), jnp.float32),
        pltpu.VMEM((2, C, D), jnp.float32),
        pltpu.VMEM((C // 2 * L,), jnp.float32),
        pltpu.VMEM((BPW,), jnp.float32),
        pltpu.SemaphoreType.DMA,
        pltpu.SemaphoreType.DMA,
    ],
)
def _complex_score_sc(h_hbm, r_hbm, t_hbm, ent_hbm, rel_hbm, out_hbm,
                      idx_h, idx_r, idx_t, rows_h, rows_r, rows_t,
                      pairbuf, outb, sem0, sem1):
    wid = lax.axis_index("s") * NC + lax.axis_index("c")
    base = wid * BPW

    lane = lax.iota(jnp.int32, L)
    inv = ((lane & 1) << 3) | ((lane & 2) << 1) | ((lane & 4) >> 1) | ((lane & 8) >> 3)
    fold_idx = {blk: lane ^ blk for blk in (8, 4, 2, 1)}
    merge_mask = {blk: (lane & blk) == 0 for blk in (8, 4, 2, 1)}
    sems = (sem0, sem1)

    f = CHUNKS[0][1]
    first = (
        pltpu.async_copy(h_hbm.at[pl.ds(base, f)], idx_h.at[pl.ds(0, f)], sem0),
        pltpu.async_copy(r_hbm.at[pl.ds(base, f)], idx_r.at[pl.ds(0, f)], sem0),
        pltpu.async_copy(t_hbm.at[pl.ds(base, f)], idx_t.at[pl.ds(0, f)], sem0),
    )
    for cp in first:
        cp.wait()

    def issue(ci):
        b = ci % 2
        co, size = CHUNKS[ci]
        s = sems[b]
        return (
            pltpu.async_copy(ent_hbm.at[idx_h.at[pl.ds(co, size)]],
                             rows_h.at[b, pl.ds(0, size)], s),
            pltpu.async_copy(rel_hbm.at[idx_r.at[pl.ds(co, size)]],
                             rows_r.at[b, pl.ds(0, size)], s),
            pltpu.async_copy(ent_hbm.at[idx_t.at[pl.ds(co, size)]],
                             rows_t.at[b, pl.ds(0, size)], s),
        )

    pending = [issue(0)]

    rest = (
        pltpu.async_copy(h_hbm.at[pl.ds(base + f, BPW - f)],
                         idx_h.at[pl.ds(f, BPW - f)], sem1),
        pltpu.async_copy(r_hbm.at[pl.ds(base + f, BPW - f)],
                         idx_r.at[pl.ds(f, BPW - f)], sem1),
        pltpu.async_copy(t_hbm.at[pl.ds(base + f, BPW - f)],
                         idx_t.at[pl.ds(f, BPW - f)], sem1),
    )
    for cp in rest:
        cp.wait()

    for ci in range(len(CHUNKS)):
        b = ci % 2
        co, size = CHUNKS[ci]
        if ci + 1 < len(CHUNKS):
            pending.append(issue(ci + 1))
        for cp in pending.pop(0):
            cp.wait()

        def row_acc(ro, b=b):
            acc = jnp.zeros((L,), jnp.float32)
            for j in range(HALF // L):
                hr = rows_h[b, ro, pl.ds(j * L, L)]
                hi = rows_h[b, ro, pl.ds(HALF + j * L, L)]
                rr = rows_r[b, ro, pl.ds(j * L, L)]
                ri = rows_r[b, ro, pl.ds(HALF + j * L, L)]
                tr = rows_t[b, ro, pl.ds(j * L, L)]
                ti = rows_t[b, ro, pl.ds(HALF + j * L, L)]
                a = hr * rr - hi * ri
                bb = hr * ri + hi * rr
                acc = acc + a * tr + bb * ti
            return acc

        def pair_body(qq, _, b=b):
            for u in range(4):
                q = 4 * qq + u
                a_v = row_acc(2 * q, b)
                b_v = row_acc(2 * q + 1, b)
                fa = a_v + _perm(a_v, fold_idx[L // 2])
                fb = b_v + _perm(b_v, fold_idx[L // 2])
                pairbuf[pl.ds(q * L, L)] = jnp.where(merge_mask[L // 2], fa, fb)
            return 0

        def group_body(g, _, co=co):
            stack = []
            for q in range(L // 2):
                stack.append((1, pairbuf[pl.ds((g * (L // 2) + q) * L, L)]))
                while len(stack) >= 2 and stack[-1][0] == stack[-2][0]:
                    lvl, b_v = stack.pop()
                    _, a_v = stack.pop()
                    blk = (L // 2) >> lvl
                    fa = a_v + _perm(a_v, fold_idx[blk])
                    fb = b_v + _perm(b_v, fold_idx[blk])
                    stack.append((lvl + 1, jnp.where(merge_mask[blk], fa, fb)))
            res = _perm(stack[0][1], inv)
            outb[pl.ds(co + g * L, L)] = res
            return 0

        lax.fori_loop(0, size // 8, pair_body, 0)
        lax.fori_loop(0, size // L, group_body, 0)

    pltpu.sync_copy(outb, out_hbm.at[pl.ds(base, BPW)])


def kernel(h, r, t, ent_table, rel_table):
    score = _complex_score_sc(h, r, t, ent_table, rel_table)
    return score[:, None]


# speedup vs baseline: 1.0438x; 1.0438x over previous
import functools

import jax
import jax.numpy as jnp
from jax import lax
from jax.experimental import pallas as pl
from jax.experimental.pallas import tpu as pltpu
from jax.experimental.pallas import tpu_sc as plsc

B = 16384
D = 128
HALF = D // 2
L = 16
NC, NS = 2, 16
NW = NC * NS
BPW = B // NW
C = 128
CHUNKS = ((0, 32), (32, 96), (128, 128), (256, 128), (384, 128))

_mesh = plsc.VectorSubcoreMesh(core_axis_name="c", subcore_axis_name="s")


_PERM_DNUMS = lax.GatherDimensionNumbers(
    offset_dims=(), collapsed_slice_dims=(0,), start_index_map=(0,))


def _perm(v, idx):
    return lax.gather(v, idx[:, None], _PERM_DNUMS, (1,),
                      mode=lax.GatherScatterMode.PROMISE_IN_BOUNDS)


@functools.partial(
    pl.kernel,
    mesh=_mesh,
    out_type=jax.ShapeDtypeStruct((B,), jnp.float32),
    scratch_types=[
        pltpu.VMEM((BPW,), jnp.int32),
        pltpu.VMEM((BPW,), jnp.int32),
        pltpu.VMEM((BPW,), jnp.int32),
        pltpu.VMEM((2, C, D), jnp.float32),
        pltpu.VMEM((2, C, D), jnp.float32),
        pltpu.VMEM((2, C, D), jnp.float32),
        pltpu.VMEM((C // 2 * L,), jnp.float32),
        pltpu.VMEM((BPW,), jnp.float32),
        pltpu.SemaphoreType.DMA,
        pltpu.SemaphoreType.DMA,
    ],
)
def _complex_score_sc(h_hbm, r_hbm, t_hbm, ent_hbm, rel_hbm, out_hbm,
                      idx_h, idx_r, idx_t, rows_h, rows_r, rows_t,
                      pairbuf, outb, sem0, sem1):
    wid = lax.axis_index("s") * NC + lax.axis_index("c")
    base = wid * BPW

    lane = lax.iota(jnp.int32, L)
    inv = ((lane & 1) << 3) | ((lane & 2) << 1) | ((lane & 4) >> 1) | ((lane & 8) >> 3)
    fold_idx = {blk: lane ^ blk for blk in (8, 4, 2, 1)}
    merge_mask = {blk: (lane & blk) == 0 for blk in (8, 4, 2, 1)}
    sems = (sem0, sem1)

    f = CHUNKS[0][1]
    first = (
        pltpu.async_copy(h_hbm.at[pl.ds(base, f)], idx_h.at[pl.ds(0, f)], sem0),
        pltpu.async_copy(r_hbm.at[pl.ds(base, f)], idx_r.at[pl.ds(0, f)], sem0),
        pltpu.async_copy(t_hbm.at[pl.ds(base, f)], idx_t.at[pl.ds(0, f)], sem0),
    )
    for cp in first:
        cp.wait()

    def issue(ci):
        b = ci % 2
        co, size = CHUNKS[ci]
        s = sems[b]
        return (
            pltpu.async_copy(ent_hbm.at[idx_h.at[pl.ds(co, size)]],
                             rows_h.at[b, pl.ds(0, size)], s),
            pltpu.async_copy(rel_hbm.at[idx_r.at[pl.ds(co, size)]],
                             rows_r.at[b, pl.ds(0, size)], s),
            pltpu.async_copy(ent_hbm.at[idx_t.at[pl.ds(co, size)]],
                             rows_t.at[b, pl.ds(0, size)], s),
        )

    pending = [issue(0)]

    rest = (
        pltpu.async_copy(h_hbm.at[pl.ds(base + f, BPW - f)],
                         idx_h.at[pl.ds(f, BPW - f)], sem1),
        pltpu.async_copy(r_hbm.at[pl.ds(base + f, BPW - f)],
                         idx_r.at[pl.ds(f, BPW - f)], sem1),
        pltpu.async_copy(t_hbm.at[pl.ds(base + f, BPW - f)],
                         idx_t.at[pl.ds(f, BPW - f)], sem1),
    )
    for cp in rest:
        cp.wait()

    for ci in range(len(CHUNKS)):
        b = ci % 2
        co, size = CHUNKS[ci]
        if ci + 1 < len(CHUNKS):
            pending.append(issue(ci + 1))
        for cp in pending.pop(0):
            cp.wait()

        def row_acc(ro, b=b):
            acc = jnp.zeros((L,), jnp.float32)
            for j in range(HALF // L):
                hr = rows_h[b, ro, pl.ds(j * L, L)]
                hi = rows_h[b, ro, pl.ds(HALF + j * L, L)]
                rr = rows_r[b, ro, pl.ds(j * L, L)]
                ri = rows_r[b, ro, pl.ds(HALF + j * L, L)]
                tr = rows_t[b, ro, pl.ds(j * L, L)]
                ti = rows_t[b, ro, pl.ds(HALF + j * L, L)]
                a = hr * rr - hi * ri
                bb = hr * ri + hi * rr
                acc = acc + a * tr + bb * ti
            return acc

        def pair_body(q, b=b):
            a_v = row_acc(2 * q, b)
            b_v = row_acc(2 * q + 1, b)
            fa = a_v + _perm(a_v, fold_idx[L // 2])
            fb = b_v + _perm(b_v, fold_idx[L // 2])
            pairbuf[pl.ds(q * L, L)] = jnp.where(merge_mask[L // 2], fa, fb)

        def group_body(g, _, co=co):
            stack = []
            for q in range(L // 2):
                stack.append((1, pairbuf[pl.ds((g * (L // 2) + q) * L, L)]))
                while len(stack) >= 2 and stack[-1][0] == stack[-2][0]:
                    lvl, b_v = stack.pop()
                    _, a_v = stack.pop()
                    blk = (L // 2) >> lvl
                    fa = a_v + _perm(a_v, fold_idx[blk])
                    fb = b_v + _perm(b_v, fold_idx[blk])
                    stack.append((lvl + 1, jnp.where(merge_mask[blk], fa, fb)))
            res = _perm(stack[0][1], inv)
            outb[pl.ds(co + g * L, L)] = res
            return 0

        plsc.parallel_loop(0, size // 2, step=1, unroll=2)(pair_body)
        lax.fori_loop(0, size // L, group_body, 0)

    pltpu.sync_copy(outb, out_hbm.at[pl.ds(base, BPW)])


def kernel(h, r, t, ent_table, rel_table):
    score = _complex_score_sc(h, r, t, ent_table, rel_table)
    return score[:, None]
